# baseline (speedup 1.0000x reference)
import jax
import jax.numpy as jnp
from jax.experimental import pallas as pl
from jax.experimental.pallas import tpu as pltpu

_LN_EPS = 1e-5
_LEAKY_SLOPE = 0.01


def _gat_tile_kernel(x_ref, gamma_ref, beta_ref, wqkv_ref, bqkv_ref, o_ref):
    bt, J, D = x_ref.shape
    M = bt * J

    x = x_ref[...].reshape(M, D)

    mu = jnp.mean(x, axis=-1, keepdims=True)
    msq = jnp.mean(x * x, axis=-1, keepdims=True)
    rstd = jax.lax.rsqrt(msq - mu * mu + _LN_EPS)
    xn = (x - mu) * rstd
    xn = xn * gamma_ref[...] + beta_ref[...]

    qkv = jnp.dot(xn, wqkv_ref[...], preferred_element_type=jnp.float32)
    qkv = qkv + bqkv_ref[...]
    qkv = qkv.reshape(bt, J, 3 * D)
    q = qkv[..., :D]
    k = qkv[..., D:2 * D]
    v = qkv[..., 2 * D:]

    s = jnp.einsum("bqd,bkd->bqk", q, k,
                   preferred_element_type=jnp.float32)
    p = jnp.exp(s)
    r = 1.0 / jnp.sum(p, axis=-1, keepdims=True)
    att = jnp.einsum("bqk,bkd->bqd", p, v,
                     preferred_element_type=jnp.float32)
    att = att * r

    act = jnp.maximum(att, _LEAKY_SLOPE * att)
    o_ref[...] = (act.reshape(M, D) + x).reshape(bt, J, D).astype(o_ref.dtype)


def kernel(x, gamma, beta, wqkv, bqkv):
    B, J, D = x.shape
    BT = 256
    grid_b = B // BT

    fixed = lambda b: (0, 0)

    return pl.pallas_call(
        _gat_tile_kernel,
        out_shape=jax.ShapeDtypeStruct((B, J, D), x.dtype),
        grid=(grid_b,),
        in_specs=[
            pl.BlockSpec((BT, J, D), lambda b: (b, 0, 0)),
            pl.BlockSpec((1, D), fixed),
            pl.BlockSpec((1, D), fixed),
            pl.BlockSpec((D, 3 * D), fixed),
            pl.BlockSpec((1, 3 * D), fixed),
        ],
        out_specs=pl.BlockSpec((BT, J, D), lambda b: (b, 0, 0)),
        compiler_params=pltpu.CompilerParams(
            dimension_semantics=("parallel",)),
    )(x, gamma, beta, wqkv, bqkv)


# speedup vs baseline: 1.1083x; 1.1083x over previous
import jax
import jax.numpy as jnp
from jax.experimental import pallas as pl
from jax.experimental.pallas import tpu as pltpu

_LN_EPS = 1e-5
_LEAKY_SLOPE = 0.01


def _gat_tile_kernel(x_ref, gamma_ref, beta_ref, wqkv_ref, bqkv_ref, o_ref):
    bt, J, D = x_ref.shape
    M = bt * J

    x = x_ref[...].reshape(M, D)

    mu = jnp.mean(x, axis=-1, keepdims=True)
    msq = jnp.mean(x * x, axis=-1, keepdims=True)
    rstd = jax.lax.rsqrt(msq - mu * mu + _LN_EPS)
    xn = (x - mu) * rstd
    xn = xn * gamma_ref[...] + beta_ref[...]

    qkv = jnp.dot(xn, wqkv_ref[...], preferred_element_type=jnp.float32)
    qkv = qkv + bqkv_ref[...]
    qkv = qkv.reshape(bt, J, 3 * D)
    q = qkv[..., :D]
    k = qkv[..., D:2 * D]
    v = qkv[..., 2 * D:]

    s = jnp.einsum("bqd,bkd->bqk", q, k,
                   preferred_element_type=jnp.float32)
    p = jnp.exp(s)
    r = 1.0 / jnp.sum(p, axis=-1, keepdims=True)
    att = jnp.einsum("bqk,bkd->bqd", p, v,
                     preferred_element_type=jnp.float32)
    att = att * r

    act = jnp.maximum(att, _LEAKY_SLOPE * att)
    o_ref[...] = (act.reshape(M, D) + x).reshape(bt, J, D).astype(o_ref.dtype)


def kernel(x, gamma, beta, wqkv, bqkv):
    B, J, D = x.shape
    BT = 128
    grid_b = B // BT

    fixed = lambda b: (0, 0)

    return pl.pallas_call(
        _gat_tile_kernel,
        out_shape=jax.ShapeDtypeStruct((B, J, D), x.dtype),
        grid=(grid_b,),
        in_specs=[
            pl.BlockSpec((BT, J, D), lambda b: (b, 0, 0)),
            pl.BlockSpec((1, D), fixed),
            pl.BlockSpec((1, D), fixed),
            pl.BlockSpec((D, 3 * D), fixed),
            pl.BlockSpec((1, 3 * D), fixed),
        ],
        out_specs=pl.BlockSpec((BT, J, D), lambda b: (b, 0, 0)),
        compiler_params=pltpu.CompilerParams(
            dimension_semantics=("parallel",)),
    )(x, gamma, beta, wqkv, bqkv)
